# baseline (speedup 1.0000x reference)
import jax
import jax.numpy as jnp
from jax.experimental import pallas as pl
from jax.experimental.pallas import tpu as pltpu

_TB = 1024


def _vq_kernel(rep_ref, cb_ref, out_ref):
    rep = rep_ref[...]
    cb = cb_ref[...]
    ab = jax.lax.dot_general(
        rep, cb, (((1,), (1,)), ((), ())),
        preferred_element_type=jnp.float32)
    a2 = jnp.sum(rep * rep, axis=1, keepdims=True)
    b2 = jnp.sum(cb * cb, axis=1)[None, :]
    dist = a2 + b2 - 2.0 * ab
    k = dist.shape[1]
    minval = jnp.min(dist, axis=1, keepdims=True)
    iota = jax.lax.broadcasted_iota(jnp.int32, dist.shape, 1)
    idx = jnp.min(jnp.where(dist == minval, iota, k), axis=1)
    out_ref[0, 0, :] = idx.astype(jnp.int32)


def kernel(rep, codebook):
    B, M, D = rep.shape
    K = codebook.shape[0]
    n = B * M
    nb = n // _TB
    rep_flat = rep.reshape(n, D)
    out = pl.pallas_call(
        _vq_kernel,
        grid=(nb,),
        in_specs=[
            pl.BlockSpec((_TB, D), lambda i: (i, 0)),
            pl.BlockSpec((K, D), lambda i: (0, 0)),
        ],
        out_specs=pl.BlockSpec((1, 1, _TB), lambda i: (i, 0, 0)),
        out_shape=jax.ShapeDtypeStruct((nb, 1, _TB), jnp.int32),
        compiler_params=pltpu.CompilerParams(
            dimension_semantics=("arbitrary",),
        ),
    )(rep_flat, codebook)
    return out.reshape(B, M)


# speedup vs baseline: 1.0026x; 1.0026x over previous
import jax
import jax.numpy as jnp
from jax.experimental import pallas as pl
from jax.experimental.pallas import tpu as pltpu

_TB = 1024


def _vq_kernel(rep_ref, cb_ref, out_ref):
    rep = rep_ref[...]
    cb = cb_ref[...]
    ab = jax.lax.dot_general(
        rep, cb, (((1,), (1,)), ((), ())),
        preferred_element_type=jnp.float32)
    a2 = jnp.sum(rep * rep, axis=1, keepdims=True)
    b2 = jnp.sum(cb * cb, axis=1)[None, :]
    dist = a2 + b2 - 2.0 * ab
    k = dist.shape[1]
    minval = jnp.min(dist, axis=1, keepdims=True)
    iota = jax.lax.broadcasted_iota(jnp.int32, dist.shape, 1)
    idx = jnp.min(jnp.where(dist == minval, iota, k), axis=1)
    out_ref[0, 0, :] = idx.astype(jnp.int32)


def kernel(rep, codebook):
    B, M, D = rep.shape
    K = codebook.shape[0]
    n = B * M
    nb = n // _TB
    rep_flat = rep.reshape(n, D)
    out = pl.pallas_call(
        _vq_kernel,
        grid=(nb,),
        in_specs=[
            pl.BlockSpec((_TB, D), lambda i: (i, 0)),
            pl.BlockSpec((K, D), lambda i: (0, 0)),
        ],
        out_specs=pl.BlockSpec((1, 1, _TB), lambda i: (i, 0, 0)),
        out_shape=jax.ShapeDtypeStruct((nb, 1, _TB), jnp.int32),
        compiler_params=pltpu.CompilerParams(
            dimension_semantics=("parallel",),
        ),
    )(rep_flat, codebook)
    return out.reshape(B, M)


# speedup vs baseline: 1.0227x; 1.0200x over previous
import jax
import jax.numpy as jnp
from jax.experimental import pallas as pl
from jax.experimental.pallas import tpu as pltpu

_TB = 1024


_LC = 128


def _vq_kernel(rep_ref, cb_ref, out_ref):
    rep = rep_ref[...]
    cb = cb_ref[...]
    k = cb.shape[0]
    a2 = jnp.sum(rep * rep, axis=1, keepdims=True)
    b2 = jnp.sum(cb * cb, axis=1)[None, :]
    ab2 = jax.lax.dot_general(
        rep * (-2.0), cb, (((1,), (1,)), ((), ())),
        preferred_element_type=jnp.float32)
    nc = k // _LC
    runval = (a2 + b2[:, 0:_LC]) + ab2[:, 0:_LC]
    runidx = jnp.zeros(runval.shape, jnp.int32)
    for c in range(1, nc):
        sl = slice(c * _LC, (c + 1) * _LC)
        dist = (a2 + b2[:, sl]) + ab2[:, sl]
        cond = dist < runval
        runval = jnp.minimum(dist, runval)
        runidx = jnp.where(cond, jnp.int32(c), runidx)
    m = jnp.min(runval, axis=1, keepdims=True)
    lane = jax.lax.broadcasted_iota(jnp.int32, runval.shape, 1)
    kidx = runidx * _LC + lane
    cand = jnp.where(runval == m, kidx, jnp.int32(k))
    out_ref[0, 0, :] = jnp.min(cand, axis=1)


def kernel(rep, codebook):
    B, M, D = rep.shape
    K = codebook.shape[0]
    n = B * M
    nb = n // _TB
    rep_flat = rep.reshape(n, D)
    out = pl.pallas_call(
        _vq_kernel,
        grid=(nb,),
        in_specs=[
            pl.BlockSpec((_TB, D), lambda i: (i, 0)),
            pl.BlockSpec((K, D), lambda i: (0, 0)),
        ],
        out_specs=pl.BlockSpec((1, 1, _TB), lambda i: (i, 0, 0)),
        out_shape=jax.ShapeDtypeStruct((nb, 1, _TB), jnp.int32),
        compiler_params=pltpu.CompilerParams(
            dimension_semantics=("parallel",),
        ),
    )(rep_flat, codebook)
    return out.reshape(B, M)


# speedup vs baseline: 1.9636x; 1.9201x over previous
import jax
import jax.numpy as jnp
from jax.experimental import pallas as pl
from jax.experimental.pallas import tpu as pltpu

_TB = 1024


_KC = 128
_SC = 8


def _vq_kernel(rep_ref, cb_ref, out_ref):
    rep = rep_ref[...]
    cb = cb_ref[...]
    k = cb.shape[0]
    d = rep.shape[1]
    rep2 = rep * (-2.0)
    ones_row = jnp.ones((1, d), jnp.float32)
    a2 = jax.lax.dot_general(
        ones_row, rep * rep, (((1,), (1,)), ((), ())),
        preferred_element_type=jnp.float32)
    b2 = jnp.sum(cb * cb, axis=1, keepdims=True)
    runval = None
    runidx = None
    for c in range(k // _KC):
        abc = jax.lax.dot_general(
            cb[c * _KC:(c + 1) * _KC, :], rep2, (((1,), (1,)), ((), ())),
            preferred_element_type=jnp.float32)
        b2c = b2[c * _KC:(c + 1) * _KC, :]
        for r in range(_KC // _SC):
            rb = r * _SC
            s = b2c[rb:rb + _SC, :] + a2
            dist = s + abc[rb:rb + _SC, :]
            if runval is None:
                runval = dist
                runidx = jnp.zeros(dist.shape, jnp.int32)
            else:
                gi = c * (_KC // _SC) + r
                cond = dist < runval
                runval = jnp.minimum(dist, runval)
                runidx = jnp.where(cond, jnp.int32(gi), runidx)
    m = jnp.min(runval, axis=0, keepdims=True)
    srow = jax.lax.broadcasted_iota(jnp.int32, runval.shape, 0)
    kidx = runidx * _SC + srow
    cand = jnp.where(runval == m, kidx, jnp.int32(k))
    out_ref[0, 0, :] = jnp.min(cand, axis=0)


def kernel(rep, codebook):
    B, M, D = rep.shape
    K = codebook.shape[0]
    n = B * M
    nb = n // _TB
    rep_flat = rep.reshape(n, D)
    out = pl.pallas_call(
        _vq_kernel,
        grid=(nb,),
        in_specs=[
            pl.BlockSpec((_TB, D), lambda i: (i, 0)),
            pl.BlockSpec((K, D), lambda i: (0, 0)),
        ],
        out_specs=pl.BlockSpec((1, 1, _TB), lambda i: (i, 0, 0)),
        out_shape=jax.ShapeDtypeStruct((nb, 1, _TB), jnp.int32),
        compiler_params=pltpu.CompilerParams(
            dimension_semantics=("parallel",),
        ),
    )(rep_flat, codebook)
    return out.reshape(B, M)


# speedup vs baseline: 2.1482x; 1.0940x over previous
import jax
import jax.numpy as jnp
from jax.experimental import pallas as pl
from jax.experimental.pallas import tpu as pltpu

_TB = 2304


_KC = 128
_SC = 8


def _vq_kernel(rep_ref, cb_ref, out_ref):
    rep = rep_ref[...]
    cb = cb_ref[...]
    k = cb.shape[0]
    d = rep.shape[1]
    rep2 = rep * (-2.0)
    ones_row = jnp.ones((1, d), jnp.float32)
    a2 = jax.lax.dot_general(
        ones_row, rep * rep, (((1,), (1,)), ((), ())),
        preferred_element_type=jnp.float32)
    b2 = jnp.sum(cb * cb, axis=1, keepdims=True)
    runval = None
    runidx = None
    for c in range(k // _KC):
        abc = jax.lax.dot_general(
            cb[c * _KC:(c + 1) * _KC, :], rep2, (((1,), (1,)), ((), ())),
            preferred_element_type=jnp.float32)
        b2c = b2[c * _KC:(c + 1) * _KC, :]
        for r in range(_KC // _SC):
            rb = r * _SC
            s = b2c[rb:rb + _SC, :] + a2
            dist = s + abc[rb:rb + _SC, :]
            if runval is None:
                runval = dist
                runidx = jnp.zeros(dist.shape, jnp.int32)
            else:
                gi = c * (_KC // _SC) + r
                cond = dist < runval
                runval = jnp.minimum(dist, runval)
                runidx = jnp.where(cond, jnp.int32(gi), runidx)
    m = jnp.min(runval, axis=0, keepdims=True)
    srow = jax.lax.broadcasted_iota(jnp.int32, runval.shape, 0)
    kidx = runidx * _SC + srow
    cand = jnp.where(runval == m, kidx, jnp.int32(k))
    out_ref[0, 0, :] = jnp.min(cand, axis=0)


def kernel(rep, codebook):
    B, M, D = rep.shape
    K = codebook.shape[0]
    n = B * M
    nb = n // _TB
    rep_flat = rep.reshape(n, D)
    out = pl.pallas_call(
        _vq_kernel,
        grid=(nb,),
        in_specs=[
            pl.BlockSpec((_TB, D), lambda i: (i, 0)),
            pl.BlockSpec((K, D), lambda i: (0, 0)),
        ],
        out_specs=pl.BlockSpec((1, 1, _TB), lambda i: (i, 0, 0)),
        out_shape=jax.ShapeDtypeStruct((nb, 1, _TB), jnp.int32),
        compiler_params=pltpu.CompilerParams(
            dimension_semantics=("parallel",),
        ),
    )(rep_flat, codebook)
    return out.reshape(B, M)


# speedup vs baseline: 2.1563x; 1.0038x over previous
import jax
import jax.numpy as jnp
from jax.experimental import pallas as pl
from jax.experimental.pallas import tpu as pltpu

_TB = 4608


_KC = 128
_SC = 8


def _vq_kernel(rep_ref, cb_ref, out_ref):
    rep = rep_ref[...]
    cb = cb_ref[...]
    k = cb.shape[0]
    d = rep.shape[1]
    rep2 = rep * (-2.0)
    ones_row = jnp.ones((1, d), jnp.float32)
    a2 = jax.lax.dot_general(
        ones_row, rep * rep, (((1,), (1,)), ((), ())),
        preferred_element_type=jnp.float32)
    b2 = jnp.sum(cb * cb, axis=1, keepdims=True)
    runval = None
    runidx = None
    for c in range(k // _KC):
        abc = jax.lax.dot_general(
            cb[c * _KC:(c + 1) * _KC, :], rep2, (((1,), (1,)), ((), ())),
            preferred_element_type=jnp.float32)
        b2c = b2[c * _KC:(c + 1) * _KC, :]
        for r in range(_KC // _SC):
            rb = r * _SC
            s = b2c[rb:rb + _SC, :] + a2
            dist = s + abc[rb:rb + _SC, :]
            if runval is None:
                runval = dist
                runidx = jnp.zeros(dist.shape, jnp.int32)
            else:
                gi = c * (_KC // _SC) + r
                cond = dist < runval
                runval = jnp.minimum(dist, runval)
                runidx = jnp.where(cond, jnp.int32(gi), runidx)
    m = jnp.min(runval, axis=0, keepdims=True)
    srow = jax.lax.broadcasted_iota(jnp.int32, runval.shape, 0)
    kidx = runidx * _SC + srow
    cand = jnp.where(runval == m, kidx, jnp.int32(k))
    out_ref[0, 0, :] = jnp.min(cand, axis=0)


def kernel(rep, codebook):
    B, M, D = rep.shape
    K = codebook.shape[0]
    n = B * M
    nb = n // _TB
    rep_flat = rep.reshape(n, D)
    out = pl.pallas_call(
        _vq_kernel,
        grid=(nb,),
        in_specs=[
            pl.BlockSpec((_TB, D), lambda i: (i, 0)),
            pl.BlockSpec((K, D), lambda i: (0, 0)),
        ],
        out_specs=pl.BlockSpec((1, 1, _TB), lambda i: (i, 0, 0)),
        out_shape=jax.ShapeDtypeStruct((nb, 1, _TB), jnp.int32),
        compiler_params=pltpu.CompilerParams(
            dimension_semantics=("parallel",),
        ),
    )(rep_flat, codebook)
    return out.reshape(B, M)
